# baseline (speedup 1.0000x reference)
import dataclasses
import functools

import jax
import jax.numpy as jnp
from jax import lax
from jax.experimental import pallas as pl
from jax.experimental.pallas import tpu as pltpu
from jax.experimental.pallas import tpu_sc as plsc

NT = 32
TPS = 16
EC = 128
QC = 80


def _sc_mesh():
    return plsc.VectorSubcoreMesh(core_axis_name="c", subcore_axis_name="s")


def _make_sc_agg(N, D, CPT, with_cnt):
    RC = 80
    nrc = N // RC
    NP = N + 8
    del with_cnt

    @functools.partial(
        pl.kernel,
        out_type=jax.ShapeDtypeStruct((2, N, D), jnp.float32),
        mesh=_sc_mesh(),
        scratch_types=[
            pltpu.VMEM((EC,), jnp.int32),
            pltpu.VMEM((EC,), jnp.int32),
            pltpu.VMEM((EC,), jnp.int32),
            pltpu.VMEM((EC,), jnp.int32),
            pltpu.VMEM((EC, D), jnp.float32),
            pltpu.VMEM((EC, D), jnp.float32),
            pltpu.VMEM_SHARED((NP, D), jnp.float32),
            pltpu.SemaphoreType.DMA,
            pltpu.SemaphoreType.DMA,
            pltpu.SemaphoreType.DMA,
        ])
    def sc_agg(h_hbm, src_hbm, dst_hbm, part_hbm, sv0, sv1, dv0, dv1,
               r0, r1, accum, semg, semi0, semi1):
        c = lax.axis_index("c")
        s = lax.axis_index("s")
        wid = c * TPS + s
        tbase = wid * CPT

        @pl.loop(0, RC)
        def _(i):
            for j in range(0, D, 16):
                r0[i, pl.ds(j, 16)] = jnp.zeros((16,), jnp.float32)

        @pl.loop(s, nrc, step=TPS)
        def _(k):
            pltpu.sync_copy(r0.at[pl.ds(0, RC)], accum.at[pl.ds(k * RC, RC)])
        plsc.subcore_barrier()

        def idx_start(j, sv, dv, semi):
            off = (tbase + j) * EC
            pltpu.make_async_copy(src_hbm.at[pl.ds(off, EC)], sv, semi).start()
            pltpu.make_async_copy(dst_hbm.at[pl.ds(off, EC)], dv, semi).start()

        def idx_wait(j, sv, dv, semi):
            off = (tbase + j) * EC
            pltpu.make_async_copy(src_hbm.at[pl.ds(off, EC)], sv, semi).wait()
            pltpu.make_async_copy(dst_hbm.at[pl.ds(off, EC)], dv, semi).wait()

        idx_start(0, sv0, dv0, semi0)
        idx_start(1, sv1, dv1, semi1)

        @pl.loop(0, CPT, step=2)
        def _(k):
            for b, sv, dv, rows, semi in ((0, sv0, dv0, r0, semi0),
                                          (1, sv1, dv1, r1, semi1)):
                j = k + b
                idx_wait(j, sv, dv, semi)
                pltpu.async_copy(h_hbm.at[sv], rows, semg).wait()
                pltpu.sync_copy(rows, accum.at[dv], add=True)

                @pl.when(j + 2 < CPT)
                def _():
                    idx_start(j + 2, sv, dv, semi)

        plsc.subcore_barrier()

        @pl.loop(s, nrc, step=TPS)
        def _(k):
            off = k * RC
            pltpu.sync_copy(accum.at[pl.ds(off, RC)],
                            part_hbm.at[c, pl.ds(off, RC)])

    return sc_agg


def _make_sc_cnt(N, E):
    nchunk = E // EC

    @functools.partial(
        pl.kernel,
        out_type=jax.ShapeDtypeStruct((NT * N,), jnp.float32),
        mesh=_sc_mesh(),
        scratch_types=[
            pltpu.VMEM((EC,), jnp.int32),
            pltpu.VMEM((N,), jnp.float32),
        ],
        compiler_params=dataclasses.replace(pltpu.CompilerParams(),
                                            needs_layout_passes=False))
    def sc_cnt(dst_hbm, cnt_hbm, dstv, hist):
        c = lax.axis_index("c")
        s = lax.axis_index("s")
        wid = c * TPS + s

        @pl.loop(0, N, step=16)
        def _(i):
            hist[pl.ds(i, 16)] = jnp.zeros((16,), jnp.float32)

        ones = jnp.ones((16,), jnp.float32)

        @pl.loop(wid, nchunk, step=NT)
        def _(j):
            pltpu.sync_copy(dst_hbm.at[pl.ds(j * EC, EC)], dstv)
            for g in range(EC // 16):
                plsc.addupdate_scatter(hist, [dstv[pl.ds(g * 16, 16)]], ones)

        pltpu.sync_copy(hist, cnt_hbm.at[pl.ds(wid * N, N)])

    return sc_cnt


def _make_sc_pred(N, D, Q):
    nchunk = Q // QC

    @functools.partial(
        pl.kernel,
        out_type=jax.ShapeDtypeStruct((Q, 16), jnp.float32),
        mesh=_sc_mesh(),
        scratch_types=[
            pltpu.VMEM((QC,), jnp.int32),
            pltpu.VMEM((QC,), jnp.int32),
            pltpu.VMEM((QC, D), jnp.float32),
            pltpu.VMEM((QC, D), jnp.float32),
            pltpu.VMEM((QC, 16), jnp.float32),
            pltpu.SemaphoreType.DMA,
        ])
    def sc_pred(h_hbm, qa_hbm, qb_hbm, pred_hbm, ia, ib, ra, rb, dots, sem):
        c = lax.axis_index("c")
        s = lax.axis_index("s")
        wid = c * TPS + s

        @pl.loop(wid, nchunk, step=NT)
        def _(j):
            base = j * QC
            pltpu.sync_copy(qa_hbm.at[pl.ds(base, QC)], ia)
            pltpu.sync_copy(qb_hbm.at[pl.ds(base, QC)], ib)
            pltpu.async_copy(h_hbm.at[ia], ra, sem).wait()
            pltpu.async_copy(h_hbm.at[ib], rb, sem).wait()

            @pl.loop(0, QC)
            def _(r):
                acc = ra[r, pl.ds(0, 16)] * rb[r, pl.ds(0, 16)]
                for k in range(1, D // 16):
                    acc = acc + ra[r, pl.ds(16 * k, 16)] * rb[r, pl.ds(16 * k, 16)]
                dots[r, :] = acc

            pltpu.sync_copy(dots, pred_hbm.at[pl.ds(base, QC)])

    return sc_pred


def _make_tc_rowsum(Q):
    def body(pp_ref, o_ref):
        o_ref[...] = jnp.sum(pp_ref[...], axis=1)

    return pl.pallas_call(body,
                          out_shape=jax.ShapeDtypeStruct((Q,), jnp.float32))


def _make_tc_layer(N, D, first, leaky):
    def body(h_ref, p_ref, ci_ref, wl_ref, bl_ref, wr_ref, g_ref, b_ref,
             o_ref, *inv_out):
        if first:
            cnt = jnp.sum(ci_ref[...], axis=1, keepdims=True)
            inv = 1.0 / jnp.maximum(cnt, 1.0)
            inv_out[0][...] = inv
        else:
            inv = ci_ref[...]
        agg = (p_ref[0] + p_ref[1]) * inv
        out = jnp.dot(agg, wl_ref[...], preferred_element_type=jnp.float32)
        out = out + jnp.dot(h_ref[...], wr_ref[...],
                            preferred_element_type=jnp.float32)
        out = out + bl_ref[...]
        nrm = jnp.sqrt(jnp.sum(out * out, axis=1, keepdims=True))
        out = out / jnp.maximum(nrm, 1e-12)
        m = jnp.mean(out, axis=0, keepdims=True)
        d = out - m
        v = jnp.mean(d * d, axis=0, keepdims=True)
        out = d * (g_ref[...] / jnp.sqrt(v + 1e-5)) + b_ref[...]
        if leaky:
            out = jnp.where(out > 0.0, out, 0.01 * out)
        o_ref[...] = out

    out_shape = [jax.ShapeDtypeStruct((N, D), jnp.float32)]
    if first:
        out_shape.append(jax.ShapeDtypeStruct((N, 1), jnp.float32))
    return pl.pallas_call(body, out_shape=out_shape)


def kernel(x, edge_index, edge_label_index, Wl0, bl0, Wr0, gamma0, beta0,
           Wl1, bl1, Wr1, gamma1, beta1, Wl2, bl2, Wr2, gamma2, beta2):
    N, D = x.shape
    E = edge_index.shape[1]
    Q = edge_label_index.shape[1]
    src, dst = edge_index[0], edge_index[1]
    qa, qb = edge_label_index[0], edge_label_index[1]

    CPT = -(-E // (NT * EC))
    CPT += CPT % 2
    EP = NT * CPT * EC
    src_p = jnp.concatenate([src, jnp.zeros((EP - E,), jnp.int32)])
    dst_p = jnp.concatenate([dst, jnp.full((EP - E,), N, jnp.int32)])

    sc_agg = _make_sc_agg(N, D, CPT, with_cnt=False)
    sc_cnt = _make_sc_cnt(N, E)
    sc_pred = _make_sc_pred(N, D, Q)

    params = [(Wl0, bl0, Wr0, gamma0, beta0), (Wl1, bl1, Wr1, gamma1, beta1),
              (Wl2, bl2, Wr2, gamma2, beta2)]

    cnt_t = sc_cnt(dst).reshape(NT, N).T
    h = x
    inv = None
    for i, (Wl, bl, Wr, g, b) in enumerate(params):
        parts = sc_agg(h, src_p, dst_p)
        ci = cnt_t if i == 0 else inv
        tc = _make_tc_layer(N, D, first=(i == 0), leaky=(i < 2))
        outs = tc(h, parts, ci, Wl, bl.reshape(1, D), Wr,
                  g.reshape(1, D), b.reshape(1, D))
        if i == 0:
            h, inv = outs
        else:
            h = outs[0]

    pp = sc_pred(h, qa, qb)
    pred = _make_tc_rowsum(Q)(pp)
    return (pred, h)


# speedup vs baseline: 1.0006x; 1.0006x over previous
import dataclasses
import functools

import jax
import jax.numpy as jnp
from jax import lax
from jax.experimental import pallas as pl
from jax.experimental.pallas import tpu as pltpu
from jax.experimental.pallas import tpu_sc as plsc

NT = 32
TPS = 16
EC = 128
QC = 80


def _sc_mesh():
    return plsc.VectorSubcoreMesh(core_axis_name="c", subcore_axis_name="s")


def _make_sc_agg(N, D, CPT, with_cnt):
    RC = 80
    nrc = N // RC
    NP = N + EC
    del with_cnt

    @functools.partial(
        pl.kernel,
        out_type=jax.ShapeDtypeStruct((2, N, D), jnp.float32),
        mesh=_sc_mesh(),
        scratch_types=[
            pltpu.VMEM((EC,), jnp.int32),
            pltpu.VMEM((EC,), jnp.int32),
            pltpu.VMEM((EC,), jnp.int32),
            pltpu.VMEM((EC,), jnp.int32),
            pltpu.VMEM((EC, D), jnp.float32),
            pltpu.VMEM((EC, D), jnp.float32),
            pltpu.VMEM_SHARED((NP, D), jnp.float32),
            pltpu.SemaphoreType.DMA,
            pltpu.SemaphoreType.DMA,
            pltpu.SemaphoreType.DMA,
        ])
    def sc_agg(h_hbm, src_hbm, dst_hbm, part_hbm, sv0, sv1, dv0, dv1,
               r0, r1, accum, semg, semi0, semi1):
        c = lax.axis_index("c")
        s = lax.axis_index("s")
        wid = c * TPS + s
        tbase = wid * CPT

        @pl.loop(0, RC)
        def _(i):
            for j in range(0, D, 16):
                r0[i, pl.ds(j, 16)] = jnp.zeros((16,), jnp.float32)

        @pl.loop(s, nrc, step=TPS)
        def _(k):
            pltpu.sync_copy(r0.at[pl.ds(0, RC)], accum.at[pl.ds(k * RC, RC)])
        plsc.subcore_barrier()

        def idx_start(j, sv, dv, semi):
            off = (tbase + j) * EC
            pltpu.make_async_copy(src_hbm.at[pl.ds(off, EC)], sv, semi).start()
            pltpu.make_async_copy(dst_hbm.at[pl.ds(off, EC)], dv, semi).start()

        def idx_wait(j, sv, dv, semi):
            off = (tbase + j) * EC
            pltpu.make_async_copy(src_hbm.at[pl.ds(off, EC)], sv, semi).wait()
            pltpu.make_async_copy(dst_hbm.at[pl.ds(off, EC)], dv, semi).wait()

        idx_start(0, sv0, dv0, semi0)
        idx_start(1, sv1, dv1, semi1)

        @pl.loop(0, CPT, step=2)
        def _(k):
            for b, sv, dv, rows, semi in ((0, sv0, dv0, r0, semi0),
                                          (1, sv1, dv1, r1, semi1)):
                j = k + b
                idx_wait(j, sv, dv, semi)
                pltpu.async_copy(h_hbm.at[sv], rows, semg).wait()
                pltpu.sync_copy(rows, accum.at[dv], add=True)

                @pl.when(j + 2 < CPT)
                def _():
                    idx_start(j + 2, sv, dv, semi)

        plsc.subcore_barrier()

        @pl.loop(s, nrc, step=TPS)
        def _(k):
            off = k * RC
            pltpu.sync_copy(accum.at[pl.ds(off, RC)],
                            part_hbm.at[c, pl.ds(off, RC)])

    return sc_agg


def _make_sc_cnt(N, E):
    nchunk = E // EC

    @functools.partial(
        pl.kernel,
        out_type=jax.ShapeDtypeStruct((NT * N,), jnp.float32),
        mesh=_sc_mesh(),
        scratch_types=[
            pltpu.VMEM((EC,), jnp.int32),
            pltpu.VMEM((N,), jnp.float32),
        ],
        compiler_params=dataclasses.replace(pltpu.CompilerParams(),
                                            needs_layout_passes=False))
    def sc_cnt(dst_hbm, cnt_hbm, dstv, hist):
        c = lax.axis_index("c")
        s = lax.axis_index("s")
        wid = c * TPS + s

        @pl.loop(0, N, step=16)
        def _(i):
            hist[pl.ds(i, 16)] = jnp.zeros((16,), jnp.float32)

        ones = jnp.ones((16,), jnp.float32)

        @pl.loop(wid, nchunk, step=NT)
        def _(j):
            pltpu.sync_copy(dst_hbm.at[pl.ds(j * EC, EC)], dstv)
            for g in range(EC // 16):
                plsc.addupdate_scatter(hist, [dstv[pl.ds(g * 16, 16)]], ones)

        pltpu.sync_copy(hist, cnt_hbm.at[pl.ds(wid * N, N)])

    return sc_cnt


def _make_sc_pred(N, D, Q):
    nchunk = Q // QC

    @functools.partial(
        pl.kernel,
        out_type=jax.ShapeDtypeStruct((Q, 16), jnp.float32),
        mesh=_sc_mesh(),
        scratch_types=[
            pltpu.VMEM((QC,), jnp.int32),
            pltpu.VMEM((QC,), jnp.int32),
            pltpu.VMEM((QC, D), jnp.float32),
            pltpu.VMEM((QC, D), jnp.float32),
            pltpu.VMEM((QC, 16), jnp.float32),
            pltpu.SemaphoreType.DMA,
        ])
    def sc_pred(h_hbm, qa_hbm, qb_hbm, pred_hbm, ia, ib, ra, rb, dots, sem):
        c = lax.axis_index("c")
        s = lax.axis_index("s")
        wid = c * TPS + s

        @pl.loop(wid, nchunk, step=NT)
        def _(j):
            base = j * QC
            pltpu.sync_copy(qa_hbm.at[pl.ds(base, QC)], ia)
            pltpu.sync_copy(qb_hbm.at[pl.ds(base, QC)], ib)
            pltpu.async_copy(h_hbm.at[ia], ra, sem).wait()
            pltpu.async_copy(h_hbm.at[ib], rb, sem).wait()

            @pl.loop(0, QC)
            def _(r):
                acc = ra[r, pl.ds(0, 16)] * rb[r, pl.ds(0, 16)]
                for k in range(1, D // 16):
                    acc = acc + ra[r, pl.ds(16 * k, 16)] * rb[r, pl.ds(16 * k, 16)]
                dots[r, :] = acc

            pltpu.sync_copy(dots, pred_hbm.at[pl.ds(base, QC)])

    return sc_pred


def _make_tc_rowsum(Q):
    def body(pp_ref, o_ref):
        o_ref[...] = jnp.sum(pp_ref[...], axis=1)

    return pl.pallas_call(body,
                          out_shape=jax.ShapeDtypeStruct((Q,), jnp.float32))


def _make_tc_layer(N, D, first, leaky):
    def body(h_ref, p_ref, ci_ref, wl_ref, bl_ref, wr_ref, g_ref, b_ref,
             o_ref, *inv_out):
        if first:
            cnt = jnp.sum(ci_ref[...], axis=1, keepdims=True)
            inv = 1.0 / jnp.maximum(cnt, 1.0)
            inv_out[0][...] = inv
        else:
            inv = ci_ref[...]
        agg = (p_ref[0] + p_ref[1]) * inv
        out = jnp.dot(agg, wl_ref[...], preferred_element_type=jnp.float32)
        out = out + jnp.dot(h_ref[...], wr_ref[...],
                            preferred_element_type=jnp.float32)
        out = out + bl_ref[...]
        nrm = jnp.sqrt(jnp.sum(out * out, axis=1, keepdims=True))
        out = out / jnp.maximum(nrm, 1e-12)
        m = jnp.mean(out, axis=0, keepdims=True)
        d = out - m
        v = jnp.mean(d * d, axis=0, keepdims=True)
        out = d * (g_ref[...] / jnp.sqrt(v + 1e-5)) + b_ref[...]
        if leaky:
            out = jnp.where(out > 0.0, out, 0.01 * out)
        o_ref[...] = out

    out_shape = [jax.ShapeDtypeStruct((N, D), jnp.float32)]
    if first:
        out_shape.append(jax.ShapeDtypeStruct((N, 1), jnp.float32))
    return pl.pallas_call(body, out_shape=out_shape)


def kernel(x, edge_index, edge_label_index, Wl0, bl0, Wr0, gamma0, beta0,
           Wl1, bl1, Wr1, gamma1, beta1, Wl2, bl2, Wr2, gamma2, beta2):
    N, D = x.shape
    E = edge_index.shape[1]
    Q = edge_label_index.shape[1]
    src, dst = edge_index[0], edge_index[1]
    qa, qb = edge_label_index[0], edge_label_index[1]

    CPT = -(-E // (NT * EC))
    CPT += CPT % 2
    EP = NT * CPT * EC
    src_p = jnp.concatenate([src, jnp.zeros((EP - E,), jnp.int32)])
    dst_p = jnp.concatenate(
        [dst, N + (jnp.arange(EP - E, dtype=jnp.int32) % EC)])

    sc_agg = _make_sc_agg(N, D, CPT, with_cnt=False)
    sc_cnt = _make_sc_cnt(N, E)
    sc_pred = _make_sc_pred(N, D, Q)

    params = [(Wl0, bl0, Wr0, gamma0, beta0), (Wl1, bl1, Wr1, gamma1, beta1),
              (Wl2, bl2, Wr2, gamma2, beta2)]

    cnt_t = sc_cnt(dst).reshape(NT, N).T
    h = x
    inv = None
    for i, (Wl, bl, Wr, g, b) in enumerate(params):
        parts = sc_agg(h, src_p, dst_p)
        ci = cnt_t if i == 0 else inv
        tc = _make_tc_layer(N, D, first=(i == 0), leaky=(i < 2))
        outs = tc(h, parts, ci, Wl, bl.reshape(1, D), Wr,
                  g.reshape(1, D), b.reshape(1, D))
        if i == 0:
            h, inv = outs
        else:
            h = outs[0]

    pp = sc_pred(h, qa, qb)
    pred = _make_tc_rowsum(Q)(pp)
    return (pred, h)


# speedup vs baseline: 2.4955x; 2.4941x over previous
import dataclasses
import functools

import jax
import jax.numpy as jnp
from jax import lax
from jax.experimental import pallas as pl
from jax.experimental.pallas import tpu as pltpu
from jax.experimental.pallas import tpu_sc as plsc

NT = 32
TPS = 16
EC = 128
QC = 80


def _sc_mesh():
    return plsc.VectorSubcoreMesh(core_axis_name="c", subcore_axis_name="s")


def _make_sc_agg(N, D, CPT, with_cnt):
    RC = 80
    nrc = N // RC
    NP = N + EC
    del with_cnt

    @functools.partial(
        pl.kernel,
        out_type=jax.ShapeDtypeStruct((2, N, D), jnp.float32),
        mesh=_sc_mesh(),
        scratch_types=[
            pltpu.VMEM((EC,), jnp.int32),
            pltpu.VMEM((EC,), jnp.int32),
            pltpu.VMEM((EC,), jnp.int32),
            pltpu.VMEM((EC,), jnp.int32),
            pltpu.VMEM((EC, D), jnp.float32),
            pltpu.VMEM((EC, D), jnp.float32),
            pltpu.VMEM_SHARED((NP, D), jnp.float32),
            pltpu.SemaphoreType.DMA,
            pltpu.SemaphoreType.DMA,
            pltpu.SemaphoreType.DMA,
        ])
    def sc_agg(h_hbm, src_hbm, dst_hbm, part_hbm, sv0, sv1, dv0, dv1,
               r0, r1, accum, semg, semi0, semi1):
        c = lax.axis_index("c")
        s = lax.axis_index("s")
        wid = c * TPS + s
        tbase = wid * CPT

        @pl.loop(0, RC)
        def _(i):
            for j in range(0, D, 16):
                r0[i, pl.ds(j, 16)] = jnp.zeros((16,), jnp.float32)

        @pl.loop(s, nrc, step=TPS)
        def _(k):
            pltpu.sync_copy(r0.at[pl.ds(0, RC)], accum.at[pl.ds(k * RC, RC)])
        plsc.subcore_barrier()

        def idx_start(j, sv, dv, semi):
            off = (tbase + j) * EC
            pltpu.make_async_copy(src_hbm.at[pl.ds(off, EC)], sv, semi).start()
            pltpu.make_async_copy(dst_hbm.at[pl.ds(off, EC)], dv, semi).start()

        def idx_wait(j, sv, dv, semi):
            off = (tbase + j) * EC
            pltpu.make_async_copy(src_hbm.at[pl.ds(off, EC)], sv, semi).wait()
            pltpu.make_async_copy(dst_hbm.at[pl.ds(off, EC)], dv, semi).wait()

        idx_start(0, sv0, dv0, semi0)
        idx_start(1, sv1, dv1, semi1)

        @pl.loop(0, CPT, step=2)
        def _(k):
            for b, sv, dv, rows, semi in ((0, sv0, dv0, r0, semi0),
                                          (1, sv1, dv1, r1, semi1)):
                j = k + b
                idx_wait(j, sv, dv, semi)
                pltpu.async_copy(h_hbm.at[sv], rows, semg).wait()
                pltpu.sync_copy(rows, accum.at[dv], add=True)

                @pl.when(j + 2 < CPT)
                def _():
                    idx_start(j + 2, sv, dv, semi)

        plsc.subcore_barrier()

        @pl.loop(s, nrc, step=TPS)
        def _(k):
            off = k * RC
            pltpu.sync_copy(accum.at[pl.ds(off, RC)],
                            part_hbm.at[c, pl.ds(off, RC)])

    return sc_agg


def _make_sc_cnt(N, E):
    nchunk = E // EC

    @functools.partial(
        pl.kernel,
        out_type=jax.ShapeDtypeStruct((NT * N,), jnp.float32),
        mesh=_sc_mesh(),
        scratch_types=[
            pltpu.VMEM((EC,), jnp.int32),
            pltpu.VMEM((N,), jnp.float32),
        ],
        compiler_params=dataclasses.replace(pltpu.CompilerParams(),
                                            needs_layout_passes=False))
    def sc_cnt(dst_hbm, cnt_hbm, dstv, hist):
        c = lax.axis_index("c")
        s = lax.axis_index("s")
        wid = c * TPS + s

        @pl.loop(0, N, step=16)
        def _(i):
            hist[pl.ds(i, 16)] = jnp.zeros((16,), jnp.float32)

        ones = jnp.ones((16,), jnp.float32)

        @pl.loop(wid, nchunk, step=NT)
        def _(j):
            pltpu.sync_copy(dst_hbm.at[pl.ds(j * EC, EC)], dstv)
            for g in range(EC // 16):
                plsc.addupdate_scatter(hist, [dstv[pl.ds(g * 16, 16)]], ones)

        pltpu.sync_copy(hist, cnt_hbm.at[pl.ds(wid * N, N)])

    return sc_cnt


def _make_sc_pred(N, D, Q):
    nchunk = Q // QC

    @functools.partial(
        pl.kernel,
        out_type=jax.ShapeDtypeStruct((Q, 16), jnp.float32),
        mesh=_sc_mesh(),
        scratch_types=[
            pltpu.VMEM((QC,), jnp.int32),
            pltpu.VMEM((QC,), jnp.int32),
            pltpu.VMEM((QC, D), jnp.float32),
            pltpu.VMEM((QC, D), jnp.float32),
            pltpu.VMEM((QC, 16), jnp.float32),
            pltpu.SemaphoreType.DMA,
        ])
    def sc_pred(h_hbm, qa_hbm, qb_hbm, pred_hbm, ia, ib, ra, rb, dots, sem):
        c = lax.axis_index("c")
        s = lax.axis_index("s")
        wid = c * TPS + s

        @pl.loop(wid, nchunk, step=NT)
        def _(j):
            base = j * QC
            pltpu.sync_copy(qa_hbm.at[pl.ds(base, QC)], ia)
            pltpu.sync_copy(qb_hbm.at[pl.ds(base, QC)], ib)
            pltpu.async_copy(h_hbm.at[ia], ra, sem).wait()
            pltpu.async_copy(h_hbm.at[ib], rb, sem).wait()

            @pl.loop(0, QC)
            def _(r):
                acc = ra[r, pl.ds(0, 16)] * rb[r, pl.ds(0, 16)]
                for k in range(1, D // 16):
                    acc = acc + ra[r, pl.ds(16 * k, 16)] * rb[r, pl.ds(16 * k, 16)]
                dots[r, :] = acc

            pltpu.sync_copy(dots, pred_hbm.at[pl.ds(base, QC)])

    return sc_pred


def _make_tc_rowsum(Q):
    def body(pp_ref, o_ref):
        o_ref[...] = jnp.sum(pp_ref[...], axis=1)

    return pl.pallas_call(body,
                          out_shape=jax.ShapeDtypeStruct((Q,), jnp.float32))


def _make_tc_layer(N, D, first, leaky):
    def body(h_ref, p_ref, ci_ref, wl_ref, bl_ref, wr_ref, g_ref, b_ref,
             o_ref, *inv_out):
        if first:
            cnt = jnp.sum(ci_ref[...], axis=1, keepdims=True)
            inv = 1.0 / jnp.maximum(cnt, 1.0)
            inv_out[0][...] = inv
        else:
            inv = ci_ref[...]
        agg = (p_ref[0] + p_ref[1]) * inv
        out = jnp.dot(agg, wl_ref[...], preferred_element_type=jnp.float32)
        out = out + jnp.dot(h_ref[...], wr_ref[...],
                            preferred_element_type=jnp.float32)
        out = out + bl_ref[...]
        nrm = jnp.sqrt(jnp.sum(out * out, axis=1, keepdims=True))
        out = out / jnp.maximum(nrm, 1e-12)
        m = jnp.mean(out, axis=0, keepdims=True)
        d = out - m
        v = jnp.mean(d * d, axis=0, keepdims=True)
        out = d * (g_ref[...] / jnp.sqrt(v + 1e-5)) + b_ref[...]
        if leaky:
            out = jnp.where(out > 0.0, out, 0.01 * out)
        o_ref[...] = out

    out_shape = [jax.ShapeDtypeStruct((N, D), jnp.float32)]
    if first:
        out_shape.append(jax.ShapeDtypeStruct((N, 1), jnp.float32))
    return pl.pallas_call(body, out_shape=out_shape)


def kernel(x, edge_index, edge_label_index, Wl0, bl0, Wr0, gamma0, beta0,
           Wl1, bl1, Wr1, gamma1, beta1, Wl2, bl2, Wr2, gamma2, beta2):
    N, D = x.shape
    E = edge_index.shape[1]
    Q = edge_label_index.shape[1]
    src, dst = edge_index[0], edge_index[1]
    qa, qb = edge_label_index[0], edge_label_index[1]

    CPT = -(-E // (NT * EC))
    CPT += CPT % 2
    EP = NT * CPT * EC
    pad_iota = jnp.arange(EP - E, dtype=jnp.int32)
    src_p = jnp.concatenate([src, pad_iota % N])
    dst_p = jnp.concatenate([dst, N + pad_iota % EC])

    sc_agg = _make_sc_agg(N, D, CPT, with_cnt=False)
    sc_cnt = _make_sc_cnt(N, E)
    sc_pred = _make_sc_pred(N, D, Q)

    params = [(Wl0, bl0, Wr0, gamma0, beta0), (Wl1, bl1, Wr1, gamma1, beta1),
              (Wl2, bl2, Wr2, gamma2, beta2)]

    cnt_t = sc_cnt(dst).reshape(NT, N).T
    h = x
    inv = None
    for i, (Wl, bl, Wr, g, b) in enumerate(params):
        parts = sc_agg(h, src_p, dst_p)
        ci = cnt_t if i == 0 else inv
        tc = _make_tc_layer(N, D, first=(i == 0), leaky=(i < 2))
        outs = tc(h, parts, ci, Wl, bl.reshape(1, D), Wr,
                  g.reshape(1, D), b.reshape(1, D))
        if i == 0:
            h, inv = outs
        else:
            h = outs[0]

    pp = sc_pred(h, qa, qb)
    pred = _make_tc_rowsum(Q)(pp)
    return (pred, h)


# speedup vs baseline: 3.0599x; 1.2262x over previous
import dataclasses
import functools

import jax
import jax.numpy as jnp
from jax import lax
from jax.experimental import pallas as pl
from jax.experimental.pallas import tpu as pltpu
from jax.experimental.pallas import tpu_sc as plsc

NT = 32
TPS = 16
EC = 128
QC = 80


def _sc_mesh():
    return plsc.VectorSubcoreMesh(core_axis_name="c", subcore_axis_name="s")


def _make_sc_agg(N, D, CPT, with_cnt):
    RC = 80
    nrc = N // RC
    NP = N + EC
    del with_cnt

    @functools.partial(
        pl.kernel,
        out_type=jax.ShapeDtypeStruct((2, N, D), jnp.float32),
        mesh=_sc_mesh(),
        scratch_types=(
            [pltpu.VMEM((EC,), jnp.int32)] * 4 +
            [pltpu.VMEM((EC,), jnp.int32)] * 4 +
            [pltpu.VMEM((EC, D), jnp.float32)] * 2 +
            [pltpu.VMEM_SHARED((NP, D), jnp.float32)] +
            [pltpu.SemaphoreType.DMA] * 7
        ))
    def sc_agg(h_hbm, src_hbm, dst_hbm, part_hbm, sv0, sv1, sv2, sv3,
               dv0, dv1, dv2, dv3, r0, r1, accum, semg,
               semi0, semi1, semi2, semi3, sems0, sems1):
        c = lax.axis_index("c")
        s = lax.axis_index("s")
        wid = c * TPS + s
        tbase = wid * CPT
        svs = (sv0, sv1, sv2, sv3)
        dvs = (dv0, dv1, dv2, dv3)
        rs = (r0, r1)
        semis = (semi0, semi1, semi2, semi3)
        semss = (sems0, sems1)

        @pl.loop(0, RC)
        def _(i):
            for j in range(0, D, 16):
                r0[i, pl.ds(j, 16)] = jnp.zeros((16,), jnp.float32)

        @pl.loop(s, nrc, step=TPS)
        def _(k):
            pltpu.sync_copy(r0.at[pl.ds(0, RC)], accum.at[pl.ds(k * RC, RC)])
        plsc.subcore_barrier()

        def idx_start(j, b):
            off = (tbase + j) * EC
            pltpu.make_async_copy(src_hbm.at[pl.ds(off, EC)], svs[b],
                                  semis[b]).start()
            pltpu.make_async_copy(dst_hbm.at[pl.ds(off, EC)], dvs[b],
                                  semis[b]).start()

        def idx_wait(j, b):
            off = (tbase + j) * EC
            pltpu.make_async_copy(src_hbm.at[pl.ds(off, EC)], svs[b],
                                  semis[b]).wait()
            pltpu.make_async_copy(dst_hbm.at[pl.ds(off, EC)], dvs[b],
                                  semis[b]).wait()

        def sc_drain(rb):
            pltpu.make_async_copy(h_hbm.at[pl.ds(0, EC)], rs[rb],
                                  semss[rb]).wait()

        idx_start(0, 0)
        idx_start(1, 1)

        @pl.loop(0, CPT, step=4)
        def _(k):
            for b in range(4):
                j = k + b
                rb = b % 2
                idx_wait(j, b)

                @pl.when(j >= 2)
                def _():
                    sc_drain(rb)

                @pl.when(j + 2 < CPT)
                def _():
                    idx_start(j + 2, (b + 2) % 4)
                pltpu.async_copy(h_hbm.at[svs[b]], rs[rb], semg).wait()
                pltpu.async_copy(rs[rb], accum.at[dvs[b]], semss[rb],
                                 add=True)

        sc_drain(0)
        sc_drain(1)
        plsc.subcore_barrier()

        @pl.loop(s, nrc, step=TPS)
        def _(k):
            off = k * RC
            pltpu.sync_copy(accum.at[pl.ds(off, RC)],
                            part_hbm.at[c, pl.ds(off, RC)])

    return sc_agg


def _make_sc_cnt(N, E):
    nchunk = E // EC

    @functools.partial(
        pl.kernel,
        out_type=jax.ShapeDtypeStruct((NT * N,), jnp.float32),
        mesh=_sc_mesh(),
        scratch_types=[
            pltpu.VMEM((EC,), jnp.int32),
            pltpu.VMEM((N,), jnp.float32),
        ],
        compiler_params=dataclasses.replace(pltpu.CompilerParams(),
                                            needs_layout_passes=False))
    def sc_cnt(dst_hbm, cnt_hbm, dstv, hist):
        c = lax.axis_index("c")
        s = lax.axis_index("s")
        wid = c * TPS + s

        @pl.loop(0, N, step=16)
        def _(i):
            hist[pl.ds(i, 16)] = jnp.zeros((16,), jnp.float32)

        ones = jnp.ones((16,), jnp.float32)

        @pl.loop(wid, nchunk, step=NT)
        def _(j):
            pltpu.sync_copy(dst_hbm.at[pl.ds(j * EC, EC)], dstv)
            for g in range(EC // 16):
                plsc.addupdate_scatter(hist, [dstv[pl.ds(g * 16, 16)]], ones)

        pltpu.sync_copy(hist, cnt_hbm.at[pl.ds(wid * N, N)])

    return sc_cnt


def _make_sc_pred(N, D, Q):
    nchunk = Q // QC

    @functools.partial(
        pl.kernel,
        out_type=jax.ShapeDtypeStruct((Q, 16), jnp.float32),
        mesh=_sc_mesh(),
        scratch_types=[
            pltpu.VMEM((QC,), jnp.int32),
            pltpu.VMEM((QC,), jnp.int32),
            pltpu.VMEM((QC, D), jnp.float32),
            pltpu.VMEM((QC, D), jnp.float32),
            pltpu.VMEM((QC, 16), jnp.float32),
            pltpu.SemaphoreType.DMA,
        ])
    def sc_pred(h_hbm, qa_hbm, qb_hbm, pred_hbm, ia, ib, ra, rb, dots, sem):
        c = lax.axis_index("c")
        s = lax.axis_index("s")
        wid = c * TPS + s

        @pl.loop(wid, nchunk, step=NT)
        def _(j):
            base = j * QC
            pltpu.sync_copy(qa_hbm.at[pl.ds(base, QC)], ia)
            pltpu.sync_copy(qb_hbm.at[pl.ds(base, QC)], ib)
            pltpu.async_copy(h_hbm.at[ia], ra, sem).wait()
            pltpu.async_copy(h_hbm.at[ib], rb, sem).wait()

            @pl.loop(0, QC)
            def _(r):
                acc = ra[r, pl.ds(0, 16)] * rb[r, pl.ds(0, 16)]
                for k in range(1, D // 16):
                    acc = acc + ra[r, pl.ds(16 * k, 16)] * rb[r, pl.ds(16 * k, 16)]
                dots[r, :] = acc

            pltpu.sync_copy(dots, pred_hbm.at[pl.ds(base, QC)])

    return sc_pred


def _make_tc_rowsum(Q):
    def body(pp_ref, o_ref):
        o_ref[...] = jnp.sum(pp_ref[...], axis=1)

    return pl.pallas_call(body,
                          out_shape=jax.ShapeDtypeStruct((Q,), jnp.float32))


def _make_tc_layer(N, D, first, leaky):
    def body(h_ref, p_ref, ci_ref, wl_ref, bl_ref, wr_ref, g_ref, b_ref,
             o_ref, *inv_out):
        if first:
            cnt = jnp.sum(ci_ref[...], axis=1, keepdims=True)
            inv = 1.0 / jnp.maximum(cnt, 1.0)
            inv_out[0][...] = inv
        else:
            inv = ci_ref[...]
        agg = (p_ref[0] + p_ref[1]) * inv
        out = jnp.dot(agg, wl_ref[...], preferred_element_type=jnp.float32)
        out = out + jnp.dot(h_ref[...], wr_ref[...],
                            preferred_element_type=jnp.float32)
        out = out + bl_ref[...]
        nrm = jnp.sqrt(jnp.sum(out * out, axis=1, keepdims=True))
        out = out / jnp.maximum(nrm, 1e-12)
        m = jnp.mean(out, axis=0, keepdims=True)
        d = out - m
        v = jnp.mean(d * d, axis=0, keepdims=True)
        out = d * (g_ref[...] / jnp.sqrt(v + 1e-5)) + b_ref[...]
        if leaky:
            out = jnp.where(out > 0.0, out, 0.01 * out)
        o_ref[...] = out

    out_shape = [jax.ShapeDtypeStruct((N, D), jnp.float32)]
    if first:
        out_shape.append(jax.ShapeDtypeStruct((N, 1), jnp.float32))
    return pl.pallas_call(body, out_shape=out_shape)


def kernel(x, edge_index, edge_label_index, Wl0, bl0, Wr0, gamma0, beta0,
           Wl1, bl1, Wr1, gamma1, beta1, Wl2, bl2, Wr2, gamma2, beta2):
    N, D = x.shape
    E = edge_index.shape[1]
    Q = edge_label_index.shape[1]
    src, dst = edge_index[0], edge_index[1]
    qa, qb = edge_label_index[0], edge_label_index[1]

    CPT = -(-E // (NT * EC))
    CPT += (-CPT) % 4
    EP = NT * CPT * EC
    pad_iota = jnp.arange(EP - E, dtype=jnp.int32)
    src_p = jnp.concatenate([src, pad_iota % N])
    dst_p = jnp.concatenate([dst, N + pad_iota % EC])

    sc_agg = _make_sc_agg(N, D, CPT, with_cnt=False)
    sc_cnt = _make_sc_cnt(N, E)
    sc_pred = _make_sc_pred(N, D, Q)

    params = [(Wl0, bl0, Wr0, gamma0, beta0), (Wl1, bl1, Wr1, gamma1, beta1),
              (Wl2, bl2, Wr2, gamma2, beta2)]

    cnt_t = sc_cnt(dst).reshape(NT, N).T
    h = x
    inv = None
    for i, (Wl, bl, Wr, g, b) in enumerate(params):
        parts = sc_agg(h, src_p, dst_p)
        ci = cnt_t if i == 0 else inv
        tc = _make_tc_layer(N, D, first=(i == 0), leaky=(i < 2))
        outs = tc(h, parts, ci, Wl, bl.reshape(1, D), Wr,
                  g.reshape(1, D), b.reshape(1, D))
        if i == 0:
            h, inv = outs
        else:
            h = outs[0]

    pp = sc_pred(h, qa, qb)
    pred = _make_tc_rowsum(Q)(pp)
    return (pred, h)


# speedup vs baseline: 3.3151x; 1.0834x over previous
import dataclasses
import functools

import jax
import jax.numpy as jnp
from jax import lax
from jax.experimental import pallas as pl
from jax.experimental.pallas import tpu as pltpu
from jax.experimental.pallas import tpu_sc as plsc

NT = 32
TPS = 16
EC = 128
QC = 80


def _sc_mesh():
    return plsc.VectorSubcoreMesh(core_axis_name="c", subcore_axis_name="s")


def _make_sc_agg(N, D, CPT, with_cnt):
    RC = 80
    nrc = N // RC
    NP = N + EC
    del with_cnt

    @functools.partial(
        pl.kernel,
        out_type=jax.ShapeDtypeStruct((2, N, D), jnp.float32),
        mesh=_sc_mesh(),
        scratch_types=(
            [pltpu.VMEM((EC,), jnp.int32)] * 4 +
            [pltpu.VMEM((EC,), jnp.int32)] * 4 +
            [pltpu.VMEM((EC, D), jnp.float32)] * 2 +
            [pltpu.VMEM_SHARED((NP, D), jnp.float32)] +
            [pltpu.SemaphoreType.DMA] * 7
        ))
    def sc_agg(h_hbm, src_hbm, dst_hbm, part_hbm, sv0, sv1, sv2, sv3,
               dv0, dv1, dv2, dv3, r0, r1, accum, semg,
               semi0, semi1, semi2, semi3, sems0, sems1):
        c = lax.axis_index("c")
        s = lax.axis_index("s")
        wid = c * TPS + s
        tbase = wid * CPT
        svs = (sv0, sv1, sv2, sv3)
        dvs = (dv0, dv1, dv2, dv3)
        rs = (r0, r1)
        semis = (semi0, semi1, semi2, semi3)
        semss = (sems0, sems1)

        @pl.loop(0, RC)
        def _(i):
            for j in range(0, D, 16):
                r0[i, pl.ds(j, 16)] = jnp.zeros((16,), jnp.float32)

        @pl.loop(s, nrc, step=TPS)
        def _(k):
            pltpu.sync_copy(r0.at[pl.ds(0, RC)], accum.at[pl.ds(k * RC, RC)])
        plsc.subcore_barrier()

        def idx_start(j, b):
            off = (tbase + j) * EC
            pltpu.make_async_copy(src_hbm.at[pl.ds(off, EC)], svs[b],
                                  semis[b]).start()
            pltpu.make_async_copy(dst_hbm.at[pl.ds(off, EC)], dvs[b],
                                  semis[b]).start()

        def idx_wait(j, b):
            off = (tbase + j) * EC
            pltpu.make_async_copy(src_hbm.at[pl.ds(off, EC)], svs[b],
                                  semis[b]).wait()
            pltpu.make_async_copy(dst_hbm.at[pl.ds(off, EC)], dvs[b],
                                  semis[b]).wait()

        def sc_drain(rb):
            pltpu.make_async_copy(h_hbm.at[pl.ds(0, EC)], rs[rb],
                                  semss[rb]).wait()

        idx_start(0, 0)
        idx_start(1, 1)

        @pl.loop(0, CPT, step=4)
        def _(k):
            for b in range(4):
                j = k + b
                rb = b % 2
                idx_wait(j, b)

                @pl.when(j >= 2)
                def _():
                    sc_drain(rb)

                @pl.when(j + 2 < CPT)
                def _():
                    idx_start(j + 2, (b + 2) % 4)
                pltpu.async_copy(h_hbm.at[svs[b]], rs[rb], semg).wait()
                pltpu.async_copy(rs[rb], accum.at[dvs[b]], semss[rb],
                                 add=True)

        sc_drain(0)
        sc_drain(1)
        plsc.subcore_barrier()

        @pl.loop(s, nrc, step=TPS)
        def _(k):
            off = k * RC
            pltpu.sync_copy(accum.at[pl.ds(off, RC)],
                            part_hbm.at[c, pl.ds(off, RC)])

    return sc_agg


def _make_sc_cnt(N, E):
    nchunk = E // EC

    @functools.partial(
        pl.kernel,
        out_type=jax.ShapeDtypeStruct((NT * N,), jnp.float32),
        mesh=_sc_mesh(),
        scratch_types=[
            pltpu.VMEM((EC,), jnp.int32),
            pltpu.VMEM((N,), jnp.float32),
        ],
        compiler_params=dataclasses.replace(pltpu.CompilerParams(),
                                            needs_layout_passes=False))
    def sc_cnt(dst_hbm, cnt_hbm, dstv, hist):
        c = lax.axis_index("c")
        s = lax.axis_index("s")
        wid = c * TPS + s

        @pl.loop(0, N, step=16)
        def _(i):
            hist[pl.ds(i, 16)] = jnp.zeros((16,), jnp.float32)

        ones = jnp.ones((16,), jnp.float32)

        @pl.loop(wid, nchunk, step=NT)
        def _(j):
            pltpu.sync_copy(dst_hbm.at[pl.ds(j * EC, EC)], dstv)
            for g in range(EC // 16):
                plsc.addupdate_scatter(hist, [dstv[pl.ds(g * 16, 16)]], ones)

        pltpu.sync_copy(hist, cnt_hbm.at[pl.ds(wid * N, N)])

    return sc_cnt


def _make_sc_pred(N, D, QP, CPTQ):
    @functools.partial(
        pl.kernel,
        out_type=jax.ShapeDtypeStruct((QP, 16), jnp.float32),
        mesh=_sc_mesh(),
        scratch_types=(
            [pltpu.VMEM((QC,), jnp.int32)] * 8 +
            [pltpu.VMEM((QC, D), jnp.float32)] * 4 +
            [pltpu.VMEM((QC, 16), jnp.float32)] +
            [pltpu.SemaphoreType.DMA] * 6
        ))
    def sc_pred(h_hbm, qa_hbm, qb_hbm, pred_hbm, ia0, ia1, ia2, ia3,
                ib0, ib1, ib2, ib3, ra0, ra1, rb0, rb1, dots,
                semi0, semi1, semi2, semi3, semg0, semg1):
        c = lax.axis_index("c")
        s = lax.axis_index("s")
        wid = c * TPS + s
        tbase = wid * CPTQ
        ias = (ia0, ia1, ia2, ia3)
        ibs = (ib0, ib1, ib2, ib3)
        ras = (ra0, ra1)
        rbs = (rb0, rb1)
        semis = (semi0, semi1, semi2, semi3)
        semgs = (semg0, semg1)

        def idx_start(j, b):
            off = (tbase + j) * QC
            pltpu.make_async_copy(qa_hbm.at[pl.ds(off, QC)], ias[b],
                                  semis[b]).start()
            pltpu.make_async_copy(qb_hbm.at[pl.ds(off, QC)], ibs[b],
                                  semis[b]).start()

        def idx_wait(j, b):
            off = (tbase + j) * QC
            pltpu.make_async_copy(qa_hbm.at[pl.ds(off, QC)], ias[b],
                                  semis[b]).wait()
            pltpu.make_async_copy(qb_hbm.at[pl.ds(off, QC)], ibs[b],
                                  semis[b]).wait()

        def gather_start(b, rb):
            pltpu.async_copy(h_hbm.at[ias[b]], ras[rb], semgs[rb])
            pltpu.async_copy(h_hbm.at[ibs[b]], rbs[rb], semgs[rb])

        def gather_wait(b, rb):
            pltpu.make_async_copy(h_hbm.at[ias[b]], ras[rb],
                                  semgs[rb]).wait()
            pltpu.make_async_copy(h_hbm.at[ibs[b]], rbs[rb],
                                  semgs[rb]).wait()

        def compute_out(j, rb):
            ra, rbuf = ras[rb], rbs[rb]

            @pl.loop(0, QC)
            def _(r):
                acc = ra[r, pl.ds(0, 16)] * rbuf[r, pl.ds(0, 16)]
                for k in range(1, D // 16):
                    acc = acc + (ra[r, pl.ds(16 * k, 16)] *
                                 rbuf[r, pl.ds(16 * k, 16)])
                dots[r, :] = acc

            pltpu.sync_copy(dots, pred_hbm.at[pl.ds((tbase + j) * QC, QC)])

        idx_start(0, 0)
        idx_start(1, 1)
        idx_start(2, 2)

        @pl.loop(0, CPTQ, step=4)
        def _(k):
            for b in range(4):
                j = k + b
                rb = b % 2
                idx_wait(j, b)
                gather_start(b, rb)

                @pl.when(j >= 1)
                def _():
                    gather_wait((b + 3) % 4, 1 - rb)
                    compute_out(j - 1, 1 - rb)

                @pl.when(j + 3 < CPTQ)
                def _():
                    idx_start(j + 3, (b + 3) % 4)

        gather_wait((CPTQ - 1) % 4, (CPTQ - 1) % 2)
        compute_out(CPTQ - 1, (CPTQ - 1) % 2)

    return sc_pred


def _make_tc_rowsum(Q):
    def body(pp_ref, o_ref):
        o_ref[...] = jnp.sum(pp_ref[...], axis=1)

    return pl.pallas_call(body,
                          out_shape=jax.ShapeDtypeStruct((Q,), jnp.float32))


def _make_tc_layer(N, D, first, leaky):
    def body(h_ref, p_ref, ci_ref, wl_ref, bl_ref, wr_ref, g_ref, b_ref,
             o_ref, *inv_out):
        if first:
            cnt = jnp.sum(ci_ref[...], axis=1, keepdims=True)
            inv = 1.0 / jnp.maximum(cnt, 1.0)
            inv_out[0][...] = inv
        else:
            inv = ci_ref[...]
        agg = (p_ref[0] + p_ref[1]) * inv
        out = jnp.dot(agg, wl_ref[...], preferred_element_type=jnp.float32)
        out = out + jnp.dot(h_ref[...], wr_ref[...],
                            preferred_element_type=jnp.float32)
        out = out + bl_ref[...]
        nrm = jnp.sqrt(jnp.sum(out * out, axis=1, keepdims=True))
        out = out / jnp.maximum(nrm, 1e-12)
        m = jnp.mean(out, axis=0, keepdims=True)
        d = out - m
        v = jnp.mean(d * d, axis=0, keepdims=True)
        out = d * (g_ref[...] / jnp.sqrt(v + 1e-5)) + b_ref[...]
        if leaky:
            out = jnp.where(out > 0.0, out, 0.01 * out)
        o_ref[...] = out

    out_shape = [jax.ShapeDtypeStruct((N, D), jnp.float32)]
    if first:
        out_shape.append(jax.ShapeDtypeStruct((N, 1), jnp.float32))
    return pl.pallas_call(body, out_shape=out_shape)


def kernel(x, edge_index, edge_label_index, Wl0, bl0, Wr0, gamma0, beta0,
           Wl1, bl1, Wr1, gamma1, beta1, Wl2, bl2, Wr2, gamma2, beta2):
    N, D = x.shape
    E = edge_index.shape[1]
    Q = edge_label_index.shape[1]
    src, dst = edge_index[0], edge_index[1]
    qa, qb = edge_label_index[0], edge_label_index[1]

    CPT = -(-E // (NT * EC))
    CPT += (-CPT) % 4
    EP = NT * CPT * EC
    pad_iota = jnp.arange(EP - E, dtype=jnp.int32)
    src_p = jnp.concatenate([src, pad_iota % N])
    dst_p = jnp.concatenate([dst, N + pad_iota % EC])

    CPTQ = -(-Q // (NT * QC))
    CPTQ += (-CPTQ) % 4
    QP = NT * CPTQ * QC
    qpad = jnp.arange(QP - Q, dtype=jnp.int32) % N
    qa_p = jnp.concatenate([qa, qpad])
    qb_p = jnp.concatenate([qb, qpad])

    sc_agg = _make_sc_agg(N, D, CPT, with_cnt=False)
    sc_cnt = _make_sc_cnt(N, E)
    sc_pred = _make_sc_pred(N, D, QP, CPTQ)

    params = [(Wl0, bl0, Wr0, gamma0, beta0), (Wl1, bl1, Wr1, gamma1, beta1),
              (Wl2, bl2, Wr2, gamma2, beta2)]

    cnt_t = sc_cnt(dst).reshape(NT, N).T
    h = x
    inv = None
    for i, (Wl, bl, Wr, g, b) in enumerate(params):
        parts = sc_agg(h, src_p, dst_p)
        ci = cnt_t if i == 0 else inv
        tc = _make_tc_layer(N, D, first=(i == 0), leaky=(i < 2))
        outs = tc(h, parts, ci, Wl, bl.reshape(1, D), Wr,
                  g.reshape(1, D), b.reshape(1, D))
        if i == 0:
            h, inv = outs
        else:
            h = outs[0]

    pp = sc_pred(h, qa_p, qb_p)
    pred = _make_tc_rowsum(QP)(pp)[:Q]
    return (pred, h)
